# Initial kernel scaffold; baseline (speedup 1.0000x reference)
import jax
import jax.numpy as jnp
from jax.experimental import pallas as pl


def kernel(node_feat, edge_index, edge_attr, W1, b1, We1, be1, W2, b2, We2, be2, g1, bt1, g2, bt2):
    raise NotImplementedError("write your pallas kernel here")


# speedup vs baseline: 1.6652x; 1.6652x over previous
import functools

import jax
import jax.numpy as jnp
from jax import lax
from jax.experimental import pallas as pl
from jax.experimental.pallas import tpu as pltpu
from jax.experimental.pallas import tpu_sc as plsc

_S = 10
_NC = 2
_NS = 16
_NW = _NC * _NS
_CPW = 320
_NPAD = _NW * _CPW
_CN = 8
_NCH = _CPW // _CN
_CIDX = _CN * _S


def _gather_sums(h, ea, sidx_r, eidx_r):
    d = h.shape[1]
    mesh = plsc.VectorSubcoreMesh(core_axis_name="c", subcore_axis_name="s")

    @functools.partial(
        pl.kernel,
        mesh=mesh,
        compiler_params=pltpu.CompilerParams(use_tc_tiling_on_sc=False),
        out_type=(
            jax.ShapeDtypeStruct((_NPAD, d), jnp.float32),
            jax.ShapeDtypeStruct((_NPAD, 16), jnp.float32),
        ),
        scratch_types=[
            pltpu.VMEM((_NCH, _CIDX), jnp.int32),
            pltpu.VMEM((_NCH, _CIDX), jnp.int32),
            pltpu.VMEM((_CIDX, d), jnp.float32),
            pltpu.VMEM((_CIDX, 16), jnp.float32),
            pltpu.VMEM((_CN, d), jnp.float32),
            pltpu.VMEM((_CN, 16), jnp.float32),
            pltpu.SemaphoreType.DMA,
            pltpu.SemaphoreType.DMA,
        ],
    )
    def k(h_hbm, ea_hbm, sidx_hbm, eidx_hbm, hsum_hbm, easum_hbm,
          sidx_v, eidx_v, rows_v, earows_v, hacc_v, eaacc_v, sem1, sem2):
        wid = lax.axis_index("s") * _NC + lax.axis_index("c")
        pltpu.sync_copy(sidx_hbm.at[wid], sidx_v)
        pltpu.sync_copy(eidx_hbm.at[wid], eidx_v)

        def chunk(c, carry):
            cp1 = pltpu.async_copy(h_hbm.at[sidx_v.at[c]], rows_v, sem1)
            cp2 = pltpu.async_copy(ea_hbm.at[eidx_v.at[c]], earows_v, sem2)
            cp1.wait()
            cp2.wait()

            def node(j, carry2):
                r0 = j * _S
                for db in range(d // 16):
                    sl = pl.ds(db * 16, 16)
                    acc = rows_v[r0, sl]
                    for t in range(1, _S):
                        acc = acc + rows_v[r0 + t, sl]
                    hacc_v[j, sl] = acc
                ea_acc = earows_v[r0, :]
                for t in range(1, _S):
                    ea_acc = ea_acc + earows_v[r0 + t, :]
                eaacc_v[j, :] = ea_acc
                return carry2

            lax.fori_loop(0, _CN, node, 0)
            nb = wid * _CPW + c * _CN
            pltpu.sync_copy(hacc_v, hsum_hbm.at[pl.ds(nb, _CN)])
            pltpu.sync_copy(eaacc_v, easum_hbm.at[pl.ds(nb, _CN)])
            return carry

        lax.fori_loop(0, _NCH, chunk, 0)

    return k(h, ea, sidx_r, eidx_r)


def _dense_layer(h, hsum, easum, scale, We, be, W, b, g, bt):
    n, d_in = h.shape
    d_out = W.shape[1]
    rows = 1000
    grid = n // rows

    def body(h_ref, hs_ref, ea_ref, sc_ref, We_ref, be_ref, Wt_ref, Wb_ref,
             b_ref, g_ref, bt_ref, o_ref):
        tes = jnp.dot(ea_ref[...], We_ref[...],
                      preferred_element_type=jnp.float32)
        hn = (hs_ref[...] + tes + _S * be_ref[...]) * sc_ref[...]
        z = (jnp.dot(h_ref[...], Wt_ref[...],
                     preferred_element_type=jnp.float32)
             + jnp.dot(hn, Wb_ref[...], preferred_element_type=jnp.float32)
             + b_ref[...])
        a = jnp.maximum(z, 0.0)
        mu = jnp.mean(a, axis=-1, keepdims=True)
        xc = a - mu
        var = jnp.mean(xc * xc, axis=-1, keepdims=True)
        y = xc * lax.rsqrt(var + 1e-5) * g_ref[...] + bt_ref[...]
        o_ref[...] = jnp.maximum(y, 0.0)

    return pl.pallas_call(
        body,
        grid=(grid,),
        in_specs=[
            pl.BlockSpec((rows, d_in), lambda i: (i, 0)),
            pl.BlockSpec((rows, d_in), lambda i: (i, 0)),
            pl.BlockSpec((rows, 16), lambda i: (i, 0)),
            pl.BlockSpec((rows, 1), lambda i: (i, 0)),
            pl.BlockSpec((16, d_in), lambda i: (0, 0)),
            pl.BlockSpec((1, d_in), lambda i: (0, 0)),
            pl.BlockSpec((d_in, d_out), lambda i: (0, 0)),
            pl.BlockSpec((d_in, d_out), lambda i: (0, 0)),
            pl.BlockSpec((1, d_out), lambda i: (0, 0)),
            pl.BlockSpec((1, d_out), lambda i: (0, 0)),
            pl.BlockSpec((1, d_out), lambda i: (0, 0)),
        ],
        out_specs=pl.BlockSpec((rows, d_out), lambda i: (i, 0)),
        out_shape=jax.ShapeDtypeStruct((n, d_out), jnp.float32),
    )(h, hsum, easum, scale, We, be.reshape(1, -1), W[:d_in], W[d_in:],
      b.reshape(1, -1), g.reshape(1, -1), bt.reshape(1, -1))


def kernel(node_feat, edge_index, edge_attr, W1, b1, We1, be1,
           W2, b2, We2, be2, g1, bt1, g2, bt2):
    n = node_feat.shape[0]
    src = edge_index[0]
    dst = edge_index[1]
    deg = jnp.bincount(dst, length=n)
    order = jnp.argsort(dst)
    starts = jnp.cumsum(deg) - deg
    scale = jnp.where(deg > 0, 1.0 / _S, 0.0).astype(jnp.float32)[:, None]
    base_key = jax.random.key(42)
    pad = jnp.zeros((_NPAD - n) * _S, jnp.int32)

    h = node_feat
    layer_params = [(W1, b1, We1, be1, g1, bt1), (W2, b2, We2, be2, g2, bt2)]
    for i, (W, b, We, be, gm, bt) in enumerate(layer_params):
        u = jax.random.uniform(jax.random.fold_in(base_key, i), (n, _S))
        local = jnp.floor(u * jnp.maximum(deg, 1)[:, None]).astype(jnp.int32)
        eid = order[starts[:, None] + local]
        sidx = src[eid]
        eid_r = jnp.concatenate(
            [eid.reshape(-1).astype(jnp.int32), pad]).reshape(_NW, _NCH, _CIDX)
        sidx_r = jnp.concatenate(
            [sidx.reshape(-1).astype(jnp.int32), pad]).reshape(_NW, _NCH, _CIDX)
        hsum, easum = _gather_sums(h, edge_attr, sidx_r, eid_r)
        h = _dense_layer(h, hsum[:n], easum[:n], scale, We, be, W, b, gm, bt)
    return h


# speedup vs baseline: 1.9206x; 1.1534x over previous
import functools

import jax
import jax.numpy as jnp
from jax import lax
from jax.experimental import pallas as pl
from jax.experimental.pallas import tpu as pltpu
from jax.experimental.pallas import tpu_sc as plsc

_S = 10
_NC = 2
_NS = 16
_NW = _NC * _NS
_CPW = 320
_NPAD = _NW * _CPW
_CN = 8
_NCH = _CPW // _CN
_CIDX = _CN * _S


def _gather_sums(h, ea, sidx_r, eidx_r):
    d = h.shape[1]
    nring = 4
    ngrp = _NCH // nring
    gn = nring * _CN
    mesh = plsc.VectorSubcoreMesh(core_axis_name="c", subcore_axis_name="s")

    @functools.partial(
        pl.kernel,
        mesh=mesh,
        compiler_params=pltpu.CompilerParams(use_tc_tiling_on_sc=False),
        out_type=(
            jax.ShapeDtypeStruct((_NPAD, d), jnp.float32),
            jax.ShapeDtypeStruct((_NPAD, 16), jnp.float32),
        ),
        scratch_types=[
            pltpu.VMEM((_NCH, _CIDX), jnp.int32),
            pltpu.VMEM((_NCH, _CIDX), jnp.int32),
            [pltpu.VMEM((_CIDX, d), jnp.float32) for _ in range(nring)],
            [pltpu.VMEM((_CIDX, 16), jnp.float32) for _ in range(nring)],
            [pltpu.VMEM((gn, d), jnp.float32) for _ in range(2)],
            [pltpu.VMEM((gn, 16), jnp.float32) for _ in range(2)],
            [pltpu.SemaphoreType.DMA for _ in range(nring)],
            [pltpu.SemaphoreType.DMA for _ in range(nring)],
            [pltpu.SemaphoreType.DMA for _ in range(2)],
        ],
    )
    def k(h_hbm, ea_hbm, sidx_hbm, eidx_hbm, hsum_hbm, easum_hbm,
          sidx_v, eidx_v, rows, earows, hacc, eaacc, semh, seme, semo):
        wid = lax.axis_index("s") * _NC + lax.axis_index("c")
        pltpu.sync_copy(sidx_hbm.at[wid], sidx_v)
        pltpu.sync_copy(eidx_hbm.at[wid], eidx_v)

        def start(c, kbuf):
            pltpu.async_copy(h_hbm.at[sidx_v.at[c]], rows[kbuf], semh[kbuf])
            pltpu.async_copy(ea_hbm.at[eidx_v.at[c]], earows[kbuf], seme[kbuf])

        def wait(c, kbuf):
            pltpu.make_async_copy(
                h_hbm.at[sidx_v.at[c]], rows[kbuf], semh[kbuf]).wait()
            pltpu.make_async_copy(
                ea_hbm.at[eidx_v.at[c]], earows[kbuf], seme[kbuf]).wait()

        def compute(kbuf, p, koff):
            def node(j, carry2):
                r0 = j * _S
                jo = koff + j
                for db in range(d // 16):
                    sl = pl.ds(db * 16, 16)
                    acc = rows[kbuf][r0, sl]
                    for t in range(1, _S):
                        acc = acc + rows[kbuf][r0 + t, sl]
                    hacc[p][jo, sl] = acc
                ea_acc = earows[kbuf][r0, :]
                for t in range(1, _S):
                    ea_acc = ea_acc + earows[kbuf][r0 + t, :]
                eaacc[p][jo, :] = ea_acc
                return carry2

            lax.fori_loop(0, _CN, node, 0)

        def out_start(g, p):
            nb = wid * _CPW + g * gn
            pltpu.async_copy(hacc[p], hsum_hbm.at[pl.ds(nb, gn)], semo[p])
            pltpu.async_copy(eaacc[p], easum_hbm.at[pl.ds(nb, gn)], semo[p])

        def out_wait2(g, p):
            nb = wid * _CPW + g * gn
            pltpu.make_async_copy(
                hacc[p], hsum_hbm.at[pl.ds(nb, gn)], semo[p]).wait()
            pltpu.make_async_copy(
                eaacc[p], easum_hbm.at[pl.ds(nb, gn)], semo[p]).wait()

        for kbuf in range(nring - 1):
            start(kbuf, kbuf)

        def one_group(g, p):
            @pl.when(g >= 2)
            def _():
                out_wait2(g - 2, p)

            for kbuf in range(nring):
                c = g * nring + kbuf
                cn = c + (nring - 1)
                cn = jnp.where(cn >= _NCH, 0, cn)
                start(cn, (kbuf + nring - 1) % nring)
                wait(c, kbuf)
                compute(kbuf, p, kbuf * _CN)
            out_start(g, p)

        def group_pair(i, carry):
            one_group(2 * i, 0)
            one_group(2 * i + 1, 1)
            return carry

        lax.fori_loop(0, ngrp // 2, group_pair, 0)
        for kbuf in range(nring - 1):
            wait(0, kbuf)
        out_wait2(ngrp - 2, 0)
        out_wait2(ngrp - 1, 1)

    return k(h, ea, sidx_r, eidx_r)


def _dense_layer(h, hsum, easum, scale, We, be, W, b, g, bt):
    n, d_in = h.shape
    d_out = W.shape[1]
    rows = 1000
    grid = n // rows

    def body(h_ref, hs_ref, ea_ref, sc_ref, We_ref, be_ref, Wt_ref, Wb_ref,
             b_ref, g_ref, bt_ref, o_ref):
        tes = jnp.dot(ea_ref[...], We_ref[...],
                      preferred_element_type=jnp.float32)
        hn = (hs_ref[...] + tes + _S * be_ref[...]) * sc_ref[...]
        z = (jnp.dot(h_ref[...], Wt_ref[...],
                     preferred_element_type=jnp.float32)
             + jnp.dot(hn, Wb_ref[...], preferred_element_type=jnp.float32)
             + b_ref[...])
        a = jnp.maximum(z, 0.0)
        mu = jnp.mean(a, axis=-1, keepdims=True)
        xc = a - mu
        var = jnp.mean(xc * xc, axis=-1, keepdims=True)
        y = xc * lax.rsqrt(var + 1e-5) * g_ref[...] + bt_ref[...]
        o_ref[...] = jnp.maximum(y, 0.0)

    return pl.pallas_call(
        body,
        grid=(grid,),
        in_specs=[
            pl.BlockSpec((rows, d_in), lambda i: (i, 0)),
            pl.BlockSpec((rows, d_in), lambda i: (i, 0)),
            pl.BlockSpec((rows, 16), lambda i: (i, 0)),
            pl.BlockSpec((rows, 1), lambda i: (i, 0)),
            pl.BlockSpec((16, d_in), lambda i: (0, 0)),
            pl.BlockSpec((1, d_in), lambda i: (0, 0)),
            pl.BlockSpec((d_in, d_out), lambda i: (0, 0)),
            pl.BlockSpec((d_in, d_out), lambda i: (0, 0)),
            pl.BlockSpec((1, d_out), lambda i: (0, 0)),
            pl.BlockSpec((1, d_out), lambda i: (0, 0)),
            pl.BlockSpec((1, d_out), lambda i: (0, 0)),
        ],
        out_specs=pl.BlockSpec((rows, d_out), lambda i: (i, 0)),
        out_shape=jax.ShapeDtypeStruct((n, d_out), jnp.float32),
    )(h, hsum, easum, scale, We, be.reshape(1, -1), W[:d_in], W[d_in:],
      b.reshape(1, -1), g.reshape(1, -1), bt.reshape(1, -1))


def kernel(node_feat, edge_index, edge_attr, W1, b1, We1, be1,
           W2, b2, We2, be2, g1, bt1, g2, bt2):
    n = node_feat.shape[0]
    src = edge_index[0]
    dst = edge_index[1]
    deg = jnp.bincount(dst, length=n)
    order = jnp.argsort(dst)
    starts = jnp.cumsum(deg) - deg
    scale = jnp.where(deg > 0, 1.0 / _S, 0.0).astype(jnp.float32)[:, None]
    base_key = jax.random.key(42)
    pad = jnp.zeros((_NPAD - n) * _S, jnp.int32)

    h = node_feat
    layer_params = [(W1, b1, We1, be1, g1, bt1), (W2, b2, We2, be2, g2, bt2)]
    for i, (W, b, We, be, gm, bt) in enumerate(layer_params):
        u = jax.random.uniform(jax.random.fold_in(base_key, i), (n, _S))
        local = jnp.floor(u * jnp.maximum(deg, 1)[:, None]).astype(jnp.int32)
        eid = order[starts[:, None] + local]
        sidx = src[eid]
        eid_r = jnp.concatenate(
            [eid.reshape(-1).astype(jnp.int32), pad]).reshape(_NW, _NCH, _CIDX)
        sidx_r = jnp.concatenate(
            [sidx.reshape(-1).astype(jnp.int32), pad]).reshape(_NW, _NCH, _CIDX)
        hsum, easum = _gather_sums(h, edge_attr, sidx_r, eid_r)
        h = _dense_layer(h, hsum[:n], easum[:n], scale, We, be, W, b, gm, bt)
    return h
